# baseline (speedup 1.0000x reference)
import functools

import jax
import jax.numpy as jnp
from jax.experimental import pallas as pl
from jax.experimental.pallas import tpu as pltpu

K = 10
H = 128
DH = K * H
BLOCK_ROWS = 1000


def _cell_kernel(h_ref, c_ref, wf_ref, bf_ref, wiou_ref, biou_ref,
                 h_out_ref, c_out_ref):
    h0 = h_ref[:, 0, :].astype(jnp.bfloat16)
    u_res = jnp.dot(h0, wf_ref[0], preferred_element_type=jnp.float32)
    iou = jnp.dot(h0, wiou_ref[0], preferred_element_type=jnp.float32)
    for k in range(1, K):
        hk = h_ref[:, k, :].astype(jnp.bfloat16)
        u_res += jnp.dot(hk, wf_ref[k], preferred_element_type=jnp.float32)
        iou += jnp.dot(hk, wiou_ref[k], preferred_element_type=jnp.float32)
    u_res += bf_ref[...]
    f = jax.nn.sigmoid(u_res)
    c_red = f[:, 0:H] * c_ref[:, 0, :]
    for k in range(1, K):
        c_red += f[:, k * H:(k + 1) * H] * c_ref[:, k, :]
    iou += biou_ref[...]
    i = jax.nn.sigmoid(iou[:, 0:H])
    o = jax.nn.sigmoid(iou[:, H:2 * H])
    u = jnp.tanh(iou[:, 2 * H:3 * H])
    c_out = i * u + c_red
    c_out_ref[...] = c_out
    h_out_ref[...] = o * jnp.tanh(c_out)


@functools.partial(jax.jit, static_argnames=("interpret",))
def kernel(mailbox_h, mailbox_c, U_f_w, U_f_b, U_iou_w, b_iou,
           interpret=False):
    n = mailbox_h.shape[0]
    wf = U_f_w[:DH, :DH].astype(jnp.bfloat16).reshape(K, H, DH)
    wiou_t = U_iou_w[:, :DH].T.astype(jnp.bfloat16).reshape(K, H, 3 * H)
    bf = U_f_b[:DH].reshape(1, DH)
    grid = (pl.cdiv(n, BLOCK_ROWS),)
    h_out, c_out = pl.pallas_call(
        _cell_kernel,
        grid=grid,
        in_specs=[
            pl.BlockSpec((BLOCK_ROWS, K, H), lambda i: (i, 0, 0)),
            pl.BlockSpec((BLOCK_ROWS, K, H), lambda i: (i, 0, 0)),
            pl.BlockSpec((K, H, DH), lambda i: (0, 0, 0)),
            pl.BlockSpec((1, DH), lambda i: (0, 0)),
            pl.BlockSpec((K, H, 3 * H), lambda i: (0, 0, 0)),
            pl.BlockSpec((1, 3 * H), lambda i: (0, 0)),
        ],
        out_specs=[
            pl.BlockSpec((BLOCK_ROWS, H), lambda i: (i, 0)),
            pl.BlockSpec((BLOCK_ROWS, H), lambda i: (i, 0)),
        ],
        out_shape=[
            jax.ShapeDtypeStruct((n, H), jnp.float32),
            jax.ShapeDtypeStruct((n, H), jnp.float32),
        ],
        compiler_params=pltpu.CompilerParams(
            dimension_semantics=("arbitrary",),
        ),
        interpret=interpret,
    )(mailbox_h, mailbox_c, wf, bf, wiou_t, b_iou)
    return (h_out, c_out)


# speedup vs baseline: 1.4529x; 1.4529x over previous
import functools

import jax
import jax.numpy as jnp
from jax.experimental import pallas as pl
from jax.experimental.pallas import tpu as pltpu

K = 10
H = 128
DH = K * H
BLOCK_ROWS = 1000


def _cell_kernel(h_hbm, c_hbm, wf_ref, bf_ref, wiou_ref, biou_ref,
                 h_out_ref, c_out_ref, h_buf, c_buf, sems):
    i = pl.program_id(0)
    nb = pl.num_programs(0)
    slot = jax.lax.rem(i, 2)

    def block_copies(block, slot):
        rows = pl.ds(block * BLOCK_ROWS, BLOCK_ROWS)
        for k in range(K):
            cols = pl.ds(k * H, H)
            yield pltpu.make_async_copy(
                h_hbm.at[rows, k, :], h_buf.at[slot, :, cols],
                sems.at[slot, 0, k])
            yield pltpu.make_async_copy(
                c_hbm.at[rows, k, :], c_buf.at[slot, :, cols],
                sems.at[slot, 1, k])

    @pl.when(i == 0)
    def _():
        for cp in block_copies(0, 0):
            cp.start()

    @pl.when(i + 1 < nb)
    def _():
        for cp in block_copies(i + 1, jax.lax.rem(i + 1, 2)):
            cp.start()

    for cp in block_copies(i, slot):
        cp.wait()

    h_cat = h_buf[slot].astype(jnp.bfloat16)
    u_res = jnp.dot(h_cat, wf_ref[...],
                    preferred_element_type=jnp.float32) + bf_ref[...]
    f = jax.nn.sigmoid(u_res)
    fc = f * c_buf[slot]
    c_red = fc[:, 0:H]
    for k in range(1, K):
        c_red = c_red + fc[:, k * H:(k + 1) * H]
    iou = jnp.dot(h_cat, wiou_ref[...],
                  preferred_element_type=jnp.float32) + biou_ref[...]
    ig = jax.nn.sigmoid(iou[:, 0:H])
    og = jax.nn.sigmoid(iou[:, H:2 * H])
    ug = jnp.tanh(iou[:, 2 * H:3 * H])
    c_out = ig * ug + c_red
    c_out_ref[...] = c_out
    h_out_ref[...] = og * jnp.tanh(c_out)


@functools.partial(jax.jit, static_argnames=("interpret",))
def kernel(mailbox_h, mailbox_c, U_f_w, U_f_b, U_iou_w, b_iou,
           interpret=False):
    n = mailbox_h.shape[0]
    wf = U_f_w[:DH, :DH].astype(jnp.bfloat16)
    wiou_t = U_iou_w[:, :DH].T.astype(jnp.bfloat16)
    bf = U_f_b[:DH].reshape(1, DH)
    grid = (pl.cdiv(n, BLOCK_ROWS),)
    h_out, c_out = pl.pallas_call(
        _cell_kernel,
        grid=grid,
        in_specs=[
            pl.BlockSpec(memory_space=pltpu.MemorySpace.HBM),
            pl.BlockSpec(memory_space=pltpu.MemorySpace.HBM),
            pl.BlockSpec((DH, DH), lambda i: (0, 0)),
            pl.BlockSpec((1, DH), lambda i: (0, 0)),
            pl.BlockSpec((DH, 3 * H), lambda i: (0, 0)),
            pl.BlockSpec((1, 3 * H), lambda i: (0, 0)),
        ],
        out_specs=[
            pl.BlockSpec((BLOCK_ROWS, H), lambda i: (i, 0)),
            pl.BlockSpec((BLOCK_ROWS, H), lambda i: (i, 0)),
        ],
        out_shape=[
            jax.ShapeDtypeStruct((n, H), jnp.float32),
            jax.ShapeDtypeStruct((n, H), jnp.float32),
        ],
        scratch_shapes=[
            pltpu.VMEM((2, BLOCK_ROWS, DH), jnp.float32),
            pltpu.VMEM((2, BLOCK_ROWS, DH), jnp.float32),
            pltpu.SemaphoreType.DMA((2, 2, K)),
        ],
        compiler_params=pltpu.CompilerParams(
            dimension_semantics=("arbitrary",),
        ),
        interpret=interpret,
    )(mailbox_h, mailbox_c, wf, bf, wiou_t, b_iou)
    return (h_out, c_out)


# speedup vs baseline: 2.6013x; 1.7904x over previous
import functools

import jax
import jax.numpy as jnp
from jax.experimental import pallas as pl
from jax.experimental.pallas import tpu as pltpu

K = 10
H = 128
DH = K * H
BLOCK_ROWS = 1000


def _cell_kernel(h_ref, c_ref, wf_ref, bf_ref, wiou_ref, biou_ref,
                 h_out_ref, c_out_ref):
    h0 = h_ref[0].astype(jnp.bfloat16)
    u_res = jnp.dot(h0, wf_ref[0], preferred_element_type=jnp.float32)
    iou = jnp.dot(h0, wiou_ref[0], preferred_element_type=jnp.float32)
    for k in range(1, K):
        hk = h_ref[k].astype(jnp.bfloat16)
        u_res += jnp.dot(hk, wf_ref[k], preferred_element_type=jnp.float32)
        iou += jnp.dot(hk, wiou_ref[k], preferred_element_type=jnp.float32)
    f = jax.nn.sigmoid(u_res + bf_ref[...])
    c_red = f[:, 0:H] * c_ref[0]
    for k in range(1, K):
        c_red += f[:, k * H:(k + 1) * H] * c_ref[k]
    iou += biou_ref[...]
    ig = jax.nn.sigmoid(iou[:, 0:H])
    og = jax.nn.sigmoid(iou[:, H:2 * H])
    ug = jnp.tanh(iou[:, 2 * H:3 * H])
    c_out = ig * ug + c_red
    c_out_ref[...] = c_out
    h_out_ref[...] = og * jnp.tanh(c_out)


@functools.partial(jax.jit, static_argnames=("interpret",))
def kernel(mailbox_h, mailbox_c, U_f_w, U_f_b, U_iou_w, b_iou,
           interpret=False):
    n = mailbox_h.shape[0]
    h_t = mailbox_h.transpose(1, 0, 2)
    c_t = mailbox_c.transpose(1, 0, 2)
    wf = U_f_w[:DH, :DH].astype(jnp.bfloat16).reshape(K, H, DH)
    wiou_t = U_iou_w[:, :DH].T.astype(jnp.bfloat16).reshape(K, H, 3 * H)
    bf = U_f_b[:DH].reshape(1, DH)
    grid = (pl.cdiv(n, BLOCK_ROWS),)
    h_out, c_out = pl.pallas_call(
        _cell_kernel,
        grid=grid,
        in_specs=[
            pl.BlockSpec((K, BLOCK_ROWS, H), lambda i: (0, i, 0)),
            pl.BlockSpec((K, BLOCK_ROWS, H), lambda i: (0, i, 0)),
            pl.BlockSpec((K, H, DH), lambda i: (0, 0, 0)),
            pl.BlockSpec((1, DH), lambda i: (0, 0)),
            pl.BlockSpec((K, H, 3 * H), lambda i: (0, 0, 0)),
            pl.BlockSpec((1, 3 * H), lambda i: (0, 0)),
        ],
        out_specs=[
            pl.BlockSpec((BLOCK_ROWS, H), lambda i: (i, 0)),
            pl.BlockSpec((BLOCK_ROWS, H), lambda i: (i, 0)),
        ],
        out_shape=[
            jax.ShapeDtypeStruct((n, H), jnp.float32),
            jax.ShapeDtypeStruct((n, H), jnp.float32),
        ],
        compiler_params=pltpu.CompilerParams(
            dimension_semantics=("arbitrary",),
        ),
        interpret=interpret,
    )(h_t, c_t, wf, bf, wiou_t, b_iou)
    return (h_out, c_out)


# speedup vs baseline: 4.4411x; 1.7073x over previous
import functools

import jax
import jax.numpy as jnp
from jax.experimental import pallas as pl
from jax.experimental.pallas import tpu as pltpu

K = 10
H = 128
DH = K * H
BLOCK_ROWS = 1000


def _cell_kernel(h_ref, c_ref, wf_ref, bf_ref, wiou_ref, biou_ref,
                 h_out_ref, c_out_ref, hcat_ref):
    for k in range(K):
        hcat_ref[:, k * H:(k + 1) * H] = h_ref[k].astype(jnp.bfloat16)
    h_cat = hcat_ref[...]
    u_res = jnp.dot(h_cat, wf_ref[...], preferred_element_type=jnp.float32)
    iou = jnp.dot(h_cat, wiou_ref[...], preferred_element_type=jnp.float32)
    f = jax.nn.sigmoid(u_res + bf_ref[...])
    c_red = f[:, 0:H] * c_ref[0]
    for k in range(1, K):
        c_red += f[:, k * H:(k + 1) * H] * c_ref[k]
    iou += biou_ref[...]
    ig = jax.nn.sigmoid(iou[:, 0:H])
    og = jax.nn.sigmoid(iou[:, H:2 * H])
    ug = jnp.tanh(iou[:, 2 * H:3 * H])
    c_out = ig * ug + c_red
    c_out_ref[...] = c_out
    h_out_ref[...] = og * jnp.tanh(c_out)


@functools.partial(jax.jit, static_argnames=("interpret",))
def kernel(mailbox_h, mailbox_c, U_f_w, U_f_b, U_iou_w, b_iou,
           interpret=False):
    n = mailbox_h.shape[0]
    h_t = mailbox_h.transpose(1, 0, 2)
    c_t = mailbox_c.transpose(1, 0, 2)
    wf = U_f_w[:DH, :DH].astype(jnp.bfloat16)
    wiou_t = U_iou_w[:, :DH].T.astype(jnp.bfloat16)
    bf = U_f_b[:DH].reshape(1, DH)
    grid = (pl.cdiv(n, BLOCK_ROWS),)
    h_out, c_out = pl.pallas_call(
        _cell_kernel,
        grid=grid,
        in_specs=[
            pl.BlockSpec((K, BLOCK_ROWS, H), lambda i: (0, i, 0)),
            pl.BlockSpec((K, BLOCK_ROWS, H), lambda i: (0, i, 0)),
            pl.BlockSpec((DH, DH), lambda i: (0, 0)),
            pl.BlockSpec((1, DH), lambda i: (0, 0)),
            pl.BlockSpec((DH, 3 * H), lambda i: (0, 0)),
            pl.BlockSpec((1, 3 * H), lambda i: (0, 0)),
        ],
        out_specs=[
            pl.BlockSpec((BLOCK_ROWS, H), lambda i: (i, 0)),
            pl.BlockSpec((BLOCK_ROWS, H), lambda i: (i, 0)),
        ],
        out_shape=[
            jax.ShapeDtypeStruct((n, H), jnp.float32),
            jax.ShapeDtypeStruct((n, H), jnp.float32),
        ],
        scratch_shapes=[
            pltpu.VMEM((BLOCK_ROWS, DH), jnp.bfloat16),
        ],
        compiler_params=pltpu.CompilerParams(
            dimension_semantics=("arbitrary",),
        ),
        interpret=interpret,
    )(h_t, c_t, wf, bf, wiou_t, b_iou)
    return (h_out, c_out)
